# baseline (speedup 1.0000x reference)
import functools

import jax
import jax.numpy as jnp
from jax import lax
from jax.experimental import pallas as pl
from jax.experimental.pallas import tpu as pltpu
from jax.experimental.pallas import tpu_sc as plsc

B, C, H, W = 8, 96, 384, 384
PD = 17
RAD = PD // 2
L = 16
NC, NS = 2, 16
NW = NC * NS

PAIRS = B * C
PAIRS_W = PAIRS // NW
PP = PD * PD
SLAB_ROWS = 24
SLAB_COLS = 256
INNER = (PP + L - 1) // L
NBUF = 3
NOB = 2

_mesh = plsc.VectorSubcoreMesh(core_axis_name="c", subcore_axis_name="s")


@functools.partial(
    pl.kernel,
    out_type=jax.ShapeDtypeStruct((B, C, 24, 128), jnp.float32),
    mesh=_mesh,
    scratch_types=[
        pltpu.VMEM((L,), jnp.int32),
        pltpu.VMEM((NBUF, SLAB_ROWS, SLAB_COLS), jnp.float32),
        pltpu.VMEM((PAIRS_W, 24, 128), jnp.float32),
        pltpu.VMEM((INNER * L,), jnp.int32),
        pltpu.VMEM((INNER * L,), jnp.int32),
    ] + [pltpu.SemaphoreType.DMA] * NBUF,
    compiler_params=pltpu.CompilerParams(needs_layout_passes=False),
)
def _patch_sampler(x_hbm, centers_hbm, out_hbm,
                   centers_v, stage_v, pbuf_v, rtab_v, ctab_v, *sems):
    wid = lax.axis_index("s") * NC + lax.axis_index("c")

    pltpu.sync_copy(centers_hbm, centers_v)
    lanes = lax.iota(jnp.int32, L)
    cvec = centers_v[...]

    def fill_tabs(u, carry):
        q = jnp.minimum(u * L + lanes, PP - 1)
        r = q // PD
        rtab_v[pl.ds(u * L, L)] = r
        ctab_v[pl.ds(u * L, L)] = q - r * PD
        return carry

    lax.fori_loop(0, INNER, fill_tabs, 0)

    def bc(m):
        p_glob = wid * PAIRS_W + m
        b = p_glob // C
        return b, p_glob - b * C

    def start_copy(m):
        b, c = bc(m)
        cx = jnp.max(jnp.where(lanes == 2 * b, cvec, 0))
        cy = jnp.max(jnp.where(lanes == 2 * b + 1, cvec, 0))
        top = cy - RAD
        left = cx - RAD
        top_a = pl.multiple_of(top & ~7, 8)
        left_a = pl.multiple_of(jnp.minimum(left & ~127, W - SLAB_COLS), 128)
        cp = pltpu.async_copy(
            x_hbm.at[b, c, pl.ds(top_a, SLAB_ROWS), pl.ds(left_a, SLAB_COLS)],
            stage_v.at[m % NBUF], sems[m % NBUF])
        return cp, top - top_a, left - left_a

    pend = [start_copy(m) for m in range(NBUF - 1)]
    for m in range(PAIRS_W):
        if m + NBUF - 1 < PAIRS_W:
            pend.append(start_copy(m + NBUF - 1))
        cp, dr, dc = pend.pop(0)
        cp.wait()
        slab = stage_v.at[m % NBUF]
        pbuf = pbuf_v.at[m]
        for u in range(INNER):
            rr = rtab_v[pl.ds(u * L, L)]
            cc = ctab_v[pl.ds(u * L, L)]
            val = plsc.load_gather(slab, [dr + rr, dc + cc])
            plsc.store_scatter(pbuf, [rr, cc], val)

    b0, c0 = bc(0)
    pltpu.sync_copy(pbuf_v, out_hbm.at[b0, pl.ds(c0, PAIRS_W)])


def kernel(bchw, patch_centers, patch_diameter):
    del patch_diameter
    centers = patch_centers.astype(jnp.int32).reshape(L)
    return _patch_sampler(bchw, centers)[:, :, :PD, :PD]


# speedup vs baseline: 1.0194x; 1.0194x over previous
import functools

import jax
import jax.numpy as jnp
from jax import lax
from jax.experimental import pallas as pl
from jax.experimental.pallas import tpu as pltpu
from jax.experimental.pallas import tpu_sc as plsc

B, C, H, W = 8, 96, 384, 384
PD = 17
RAD = PD // 2
L = 16
NC, NS = 2, 16
NW = NC * NS

PAIRS = B * C
PAIRS_W = PAIRS // NW
PP = PD * PD
SLAB_ROWS = 24
SLAB_COLS = 256
INNER = (PP + L - 1) // L
NBUF = 3
NOB = 6

_mesh = plsc.VectorSubcoreMesh(core_axis_name="c", subcore_axis_name="s")


@functools.partial(
    pl.kernel,
    out_type=jax.ShapeDtypeStruct((B, C, PD, PD), jnp.float32),
    mesh=_mesh,
    scratch_types=[
        pltpu.VMEM((L,), jnp.int32),
        pltpu.VMEM((NBUF, SLAB_ROWS, SLAB_COLS), jnp.float32),
        pltpu.VMEM((NOB, PD, PD), jnp.float32),
        pltpu.VMEM((INNER * L,), jnp.int32),
        pltpu.VMEM((INNER * L,), jnp.int32),
    ] + [pltpu.SemaphoreType.DMA] * (NBUF + NOB),
    compiler_params=pltpu.CompilerParams(needs_layout_passes=False),
)
def _patch_sampler(x_hbm, centers_hbm, out_hbm,
                   centers_v, stage_v, pbuf_v, rtab_v, ctab_v, *sems):
    wid = lax.axis_index("s") * NC + lax.axis_index("c")

    pltpu.sync_copy(centers_hbm, centers_v)
    lanes = lax.iota(jnp.int32, L)
    cvec = centers_v[...]

    def fill_tabs(u, carry):
        q = jnp.minimum(u * L + lanes, PP - 1)
        r = q // PD
        rtab_v[pl.ds(u * L, L)] = r
        ctab_v[pl.ds(u * L, L)] = q - r * PD
        return carry

    lax.fori_loop(0, INNER, fill_tabs, 0)

    def bc(m):
        p_glob = wid * PAIRS_W + m
        b = p_glob // C
        return b, p_glob - b * C

    def start_copy(m):
        b, c = bc(m)
        cx = jnp.max(jnp.where(lanes == 2 * b, cvec, 0))
        cy = jnp.max(jnp.where(lanes == 2 * b + 1, cvec, 0))
        top = cy - RAD
        left = cx - RAD
        top_a = pl.multiple_of(top & ~7, 8)
        left_a = pl.multiple_of(jnp.minimum(left & ~127, W - SLAB_COLS), 128)
        cp = pltpu.async_copy(
            x_hbm.at[b, c, pl.ds(top_a, SLAB_ROWS), pl.ds(left_a, SLAB_COLS)],
            stage_v.at[m % NBUF], sems[m % NBUF])
        return cp, top - top_a, left - left_a

    pend = [start_copy(m) for m in range(NBUF - 1)]
    out_pend = []
    for m in range(PAIRS_W):
        if m + NBUF - 1 < PAIRS_W:
            pend.append(start_copy(m + NBUF - 1))
        cp, dr, dc = pend.pop(0)
        cp.wait()
        if len(out_pend) == NOB:
            out_pend.pop(0).wait()
        slab = stage_v.at[m % NBUF]
        pbuf = pbuf_v.at[m % NOB]
        for u in range(INNER):
            rr = rtab_v[pl.ds(u * L, L)]
            cc = ctab_v[pl.ds(u * L, L)]
            val = plsc.load_gather(slab, [dr + rr, dc + cc])
            plsc.store_scatter(pbuf, [rr, cc], val)
        b, c = bc(m)
        out_pend.append(
            pltpu.async_copy(pbuf, out_hbm.at[b, c], sems[NBUF + m % NOB]))
    for cp in out_pend:
        cp.wait()


def kernel(bchw, patch_centers, patch_diameter):
    del patch_diameter
    centers = patch_centers.astype(jnp.int32).reshape(L)
    return _patch_sampler(bchw, centers)
